# baseline (speedup 1.0000x reference)
import functools
import jax
import jax.numpy as jnp
from jax import lax
from jax.experimental import pallas as pl
from jax.experimental.pallas import tpu as pltpu
from jax.experimental.pallas import tpu_sc as plsc

N_ENT = 10000
N_REL = 500
D = 200
BN = 20000
EB = 160000
EG = 160000
B = 4096
NHID = 256
HID = 512

_I32 = jnp.int32
_F32 = jnp.float32


@functools.lru_cache(maxsize=None)
def _make_prop_kernel(nc, C, e_pad, n_dst, two_src):
    EBLK = e_pad // 2048
    n_out = -(-n_dst // 128) * 128
    n_zpad = n_out if n_dst < n_out else n_out + 128
    rz = n_zpad // 16
    ro = n_out // 16
    mesh = plsc.VectorSubcoreMesh(core_axis_name="c", subcore_axis_name="s")
    scratch = [
        pltpu.VMEM((EBLK, 128), _I32),
        pltpu.VMEM((EBLK, 128), _I32),
        pltpu.VMEM((EBLK, 128), _I32),
        pltpu.VMEM((128, C), _F32),
        pltpu.VMEM((128, C), _F32),
        pltpu.VMEM_SHARED((n_zpad, C), _F32),
        pltpu.SemaphoreType.DMA,
        pltpu.SemaphoreType.DMA,
    ]

    @functools.partial(
        pl.kernel, mesh=mesh,
        out_type=jax.ShapeDtypeStruct((nc * n_out, C), _F32),
        compiler_params=pltpu.CompilerParams(use_tc_tiling_on_sc=False),
        scratch_types=scratch)
    def k(*refs):
        tabs = refs[:nc]
        (srcA, srcB, dst2d, zrows, out,
         srcA_v, srcB_v, dst_v, buf0, buf1, acc, sem0, sem1) = refs[nc:]
        tid = lax.axis_index("s")
        sc = lax.axis_index("c")
        pltpu.sync_copy(dst2d.at[tid], dst_v)
        pltpu.sync_copy(srcA.at[tid], srcA_v)
        if two_src:
            pltpu.sync_copy(srcB.at[tid], srcB_v)
        for ci in range(nc):
            @pl.when(sc == ci % 2)
            def _(ci=ci):
                pltpu.sync_copy(zrows.at[pl.ds(tid * rz, rz)],
                                acc.at[pl.ds(tid * rz, rz)])
                plsc.subcore_barrier()
                tab = tabs[ci]
                s_v = srcB_v if (two_src and ci >= nc // 2) else srcA_v

                pltpu.async_copy(tab.at[s_v.at[0]], buf0, sem0)

                def step(jj, carry):
                    j = 2 * jj
                    pltpu.make_async_copy(tab.at[s_v.at[j]], buf0, sem0).wait()
                    pltpu.async_copy(tab.at[s_v.at[j + 1]], buf1, sem1)
                    pltpu.sync_copy(buf0, acc.at[dst_v.at[j]], add=True)
                    pltpu.make_async_copy(tab.at[s_v.at[j + 1]], buf1,
                                          sem1).wait()
                    pltpu.async_copy(tab.at[s_v.at[j + 2]], buf0, sem0)
                    pltpu.sync_copy(buf1, acc.at[dst_v.at[j + 1]], add=True)
                    return carry

                lax.fori_loop(0, EBLK // 2 - 1, step, 0)
                j = EBLK - 2
                pltpu.make_async_copy(tab.at[s_v.at[j]], buf0, sem0).wait()
                pltpu.async_copy(tab.at[s_v.at[j + 1]], buf1, sem1)
                pltpu.sync_copy(buf0, acc.at[dst_v.at[j]], add=True)
                pltpu.make_async_copy(tab.at[s_v.at[j + 1]], buf1, sem1).wait()
                pltpu.sync_copy(buf1, acc.at[dst_v.at[j + 1]], add=True)
                plsc.subcore_barrier()
                pltpu.sync_copy(acc.at[pl.ds(tid * ro, ro)],
                                out.at[pl.ds(ci * n_out + tid * ro, ro)])
                plsc.subcore_barrier()

    return k


def _prop(tabs, srcA, srcB, dst, n_dst, two_src=False):
    nc = len(tabs)
    C = tabs[0].shape[1]
    e_pad = dst.shape[0]
    n_out = -(-n_dst // 128) * 128
    k = _make_prop_kernel(nc, C, e_pad, n_dst, two_src)
    eblk = e_pad // 2048
    srcA2 = srcA.reshape(16, eblk, 128)
    srcB2 = srcB.reshape(16, eblk, 128)
    dst2 = dst.reshape(16, eblk, 128)
    zrows = jnp.zeros((n_out if n_dst < n_out else n_out + 128, C), _F32)
    raw = k(*tabs, srcA2, srcB2, dst2, zrows)
    return [raw[i * n_out:i * n_out + n_dst] for i in range(nc)]


@functools.lru_cache(maxsize=None)
def _make_count_kernel(e_pad, n_dst):
    EBLK = e_pad // 4096
    n_out = -(-n_dst // 128) * 128
    n_zpad = n_out if n_dst < n_out else n_out + 128
    rz = n_zpad // 16
    ro = n_out // 16
    mesh = plsc.VectorSubcoreMesh(core_axis_name="c", subcore_axis_name="s")
    scratch = [
        pltpu.VMEM((EBLK, 128), _I32),
        pltpu.VMEM((128, 16), _F32),
        pltpu.VMEM_SHARED((n_zpad, 16), _F32),
    ]

    @functools.partial(
        pl.kernel, mesh=mesh,
        out_type=jax.ShapeDtypeStruct((2 * n_out, 16), _F32),
        compiler_params=pltpu.CompilerParams(use_tc_tiling_on_sc=False),
        scratch_types=scratch)
    def k(dst2d, zrows, ones_hbm, out, dst_v, buf, acc):
        tid = lax.axis_index("s")
        sc = lax.axis_index("c")
        g = sc * 16 + tid
        pltpu.sync_copy(dst2d.at[g], dst_v)
        pltpu.sync_copy(ones_hbm, buf)
        pltpu.sync_copy(zrows.at[pl.ds(tid * rz, rz)],
                        acc.at[pl.ds(tid * rz, rz)])
        plsc.subcore_barrier()

        def step(j, carry):
            pltpu.sync_copy(buf, acc.at[dst_v.at[j]], add=True)
            return carry

        lax.fori_loop(0, EBLK, step, 0)
        plsc.subcore_barrier()
        pltpu.sync_copy(acc.at[pl.ds(tid * ro, ro)],
                        out.at[pl.ds(sc * n_out + tid * ro, ro)])

    return k


def _seg_count(dst, n_dst):
    E = dst.shape[0]
    e_pad = -(-E // 4096) * 4096
    dstp = jnp.concatenate([dst, jnp.full((e_pad - E,), n_dst, _I32)])
    n_out = -(-n_dst // 128) * 128
    k = _make_count_kernel(e_pad, n_dst)
    ones = jnp.ones((128, 16), _F32)
    zrows = jnp.zeros((n_out if n_dst < n_out else n_out + 128, 16), _F32)
    po = k(dstp.reshape(32, e_pad // 4096, 128), zrows, ones)
    return po[:n_dst, 0] + po[n_out:n_out + n_dst, 0]


def _pad_e(x, e_pad, fill):
    return jnp.concatenate([x, jnp.full((e_pad - x.shape[0],), fill, _I32)])


def _chunks(x, C):
    n, d = x.shape
    return [x[:, i * C:(i + 1) * C] for i in range(d // C)]


def _convkb_body(h_ref, r_ref, t_ref, cw_ref, cb_ref, fc2_ref, o_ref):
    h = h_ref[...]
    r = r_ref[...]
    t = t_ref[...]
    acc = jnp.zeros_like(h)
    for o in range(50):
        co = jax.nn.relu(cw_ref[o, 0] * h + cw_ref[o, 1] * r + cw_ref[o, 2] * t
                         + cb_ref[o])
        acc = acc + co * fc2_ref[o, :][None, :]
    o_ref[...] = jnp.sum(acc, axis=1, keepdims=True)


def _convkb(h, r, t, conv_w, conv_b, fc2):
    blk = 1024
    return pl.pallas_call(
        _convkb_body,
        grid=(B // blk,),
        in_specs=[
            pl.BlockSpec((blk, D), lambda i: (i, 0)),
            pl.BlockSpec((blk, D), lambda i: (i, 0)),
            pl.BlockSpec((blk, D), lambda i: (i, 0)),
            pl.BlockSpec((50, 3), lambda i: (0, 0), memory_space=pltpu.SMEM),
            pl.BlockSpec((50,), lambda i: (0,), memory_space=pltpu.SMEM),
            pl.BlockSpec((50, D), lambda i: (0, 0)),
        ],
        out_specs=pl.BlockSpec((blk, 1), lambda i: (i, 0)),
        out_shape=jax.ShapeDtypeStruct((B, 1), jnp.float32),
    )(h, r, t, conv_w, conv_b, fc2)


def kernel(*args):
    with jax.default_matmul_precision("float32"):
        return _kernel_impl(*args)


def _kernel_impl(entity_embeddings, relation_embeddings, sg1_W1, sg1_W2, sg2_W1,
                 sg2_W2, le_W, le_b, leo_W, leo_b, dgi_W, dgi_b, dgi_Wd,
                 conv_w, conv_b, fc_w, fc_b,
                 b_x, b_node_graph_index, b_edge_index, big_edge_index,
                 batch_inputs, shuf_idx):
    ent = entity_embeddings / (jnp.linalg.norm(entity_embeddings, axis=1,
                                               keepdims=True) + 1e-12)
    rel = relation_embeddings / (jnp.linalg.norm(relation_embeddings, axis=1,
                                                 keepdims=True) + 1e-12)

    b_x = b_x.astype(_I32)
    brel = b_node_graph_index.astype(_I32)
    src = b_edge_index[0].astype(_I32)
    dst = b_edge_index[1].astype(_I32)
    gsrc = big_edge_index[0].astype(_I32)
    gdst = big_edge_index[1].astype(_I32)
    shuf_idx = shuf_idx.astype(_I32)

    deg_b = jnp.maximum(_seg_count(dst, BN), 1.0)
    deg_g = jnp.maximum(_seg_count(gdst, N_ENT), 1.0)
    cnt = _seg_count(b_x, N_ENT)
    S = cnt / (cnt + 1e-16)
    inv_cnt = 1.0 / jnp.maximum(cnt, 1.0)

    eb_pad = -(-EB // 4096) * 4096
    src_p = _pad_e(src, eb_pad, 0)
    dst_p = _pad_e(dst, eb_pad, BN)
    gsrc_p = _pad_e(gsrc, eb_pad, 0)
    gdst_p = _pad_e(gdst, eb_pad, N_ENT)

    P = ent @ sg1_W1[:D]
    Rr = rel @ sg1_W1[D:]
    sx = jnp.take(shuf_idx, b_x)
    BNP = BN + 480
    ar = jnp.arange(BN, dtype=_I32)
    dum = jnp.full((BNP - BN,), BNP, _I32)
    z480 = jnp.zeros((BNP - BN,), _I32)
    dst_asm = jnp.concatenate([ar, dum, ar, dum])
    srcA_asm = jnp.concatenate([b_x, z480, N_ENT + brel, z480])
    srcB_asm = jnp.concatenate([sx, z480, N_ENT + brel, z480])
    Tc = [jnp.concatenate([pc, rc], axis=0)
          for pc, rc in zip(_chunks(P, 64), _chunks(Rr, 64))]
    xwc = _prop(Tc + Tc, srcA_asm, srcB_asm, dst_asm, BNP, two_src=True)

    p1c = _prop(xwc, src_p, src_p, dst_p, BN)
    h1 = jax.nn.relu(jnp.concatenate(p1c[:4], axis=1) / deg_b[:, None])
    h2 = jax.nn.relu(jnp.concatenate(p1c[4:], axis=1) / deg_b[:, None])
    W2p = jnp.pad(sg1_W2, ((0, 0), (0, 24)))
    hw = jnp.concatenate([h1 @ W2p, h2 @ W2p], axis=1)

    g_raw = _prop(_chunks(hw, 32), src_p, src_p, dst_p, BN)
    g12c = [c / deg_b[:, None] for c in g_raw]

    arp = _pad_e(ar, BNP, 0)
    bxp = _pad_e(b_x, BNP, N_ENT)
    o_raw = _prop(g12c, arp, arp, bxp, N_ENT)
    g1 = jnp.concatenate(g12c[:7], axis=1)[:N_ENT, :D]
    g2 = jnp.concatenate(g12c[7:], axis=1)[:N_ENT, :D]
    o1 = jnp.concatenate(o_raw[:7], axis=1)[:, :D] * inv_cnt[:, None]
    o2 = jnp.concatenate(o_raw[7:], axis=1)[:, :D] * inv_cnt[:, None]

    def finish(gh, out, base):
        return S[:, None] * (gh @ leo_W[:D] + out @ leo_W[D:]) + leo_b + base

    ec = finish(g1, o1, ent)
    ec_ = finish(g2, o2, jnp.take(ent, shuf_idx, axis=0))

    z24 = jnp.zeros((N_ENT, 24), _F32)
    e12 = jnp.concatenate([ec, z24, ec_, z24], axis=1)
    p12_raw = _prop(_chunks(e12, 32), gsrc_p, gsrc_p, gdst_p, N_ENT)
    p12c = [c / deg_g[:, None] for c in p12_raw]
    pb1 = jnp.concatenate(p12c[:7], axis=1)[:, :D]
    pb2 = jnp.concatenate(p12c[7:], axis=1)[:, :D]
    W2gp = jnp.pad(sg2_W2, ((0, 0), (0, 24)))
    hbw = jnp.concatenate([jax.nn.relu(pb1 @ sg2_W1) @ W2gp,
                           jax.nn.relu(pb2 @ sg2_W1) @ W2gp], axis=1)
    eg_raw = _prop(_chunks(hbw, 32), gsrc_p, gsrc_p, gdst_p, N_ENT)
    eg = jnp.concatenate(eg_raw[:7], axis=1)[:, :D] / deg_g[:, None]
    eg_ = jnp.concatenate(eg_raw[7:], axis=1)[:, :D] / deg_g[:, None]

    def dgi(h1, h2):
        e1 = jax.nn.relu(h1 @ dgi_W + dgi_b)
        e2 = jax.nn.relu(h2 @ dgi_W + dgi_b)
        c = jax.nn.sigmoid(jnp.mean(e1, axis=0))
        v = dgi_Wd @ c
        return jnp.concatenate([e1 @ v, e2 @ v])[None, :]

    local_logits = dgi(ec, ec_)
    global_logits = dgi(eg, eg_)

    h = jnp.take(ec, batch_inputs[:, 0], axis=0)
    r = jnp.take(rel, batch_inputs[:, 1], axis=0)
    t = jnp.take(ec, batch_inputs[:, 2], axis=0)
    out_conv = _convkb(h, r, t, conv_w, conv_b, fc_w.reshape(50, D)) + fc_b
    return (out_conv, local_logits, global_logits)


# speedup vs baseline: 1.3070x; 1.3070x over previous
import functools
import jax
import jax.numpy as jnp
from jax import lax
from jax.experimental import pallas as pl
from jax.experimental.pallas import tpu as pltpu
from jax.experimental.pallas import tpu_sc as plsc

N_ENT = 10000
N_REL = 500
D = 200
BN = 20000
EB = 160000
EG = 160000
B = 4096
NHID = 256
HID = 512

_I32 = jnp.int32
_F32 = jnp.float32


@functools.lru_cache(maxsize=None)
def _make_prop_kernel(nc, C, e_pad, n_dst, two_src):
    EBLK = e_pad // 2048
    n_out = -(-n_dst // 128) * 128
    n_zpad = n_out if n_dst < n_out else n_out + 128
    rz = n_zpad // 16
    ro = n_out // 16
    mesh = plsc.VectorSubcoreMesh(core_axis_name="c", subcore_axis_name="s")
    scratch = [
        pltpu.VMEM((EBLK, 128), _I32),
        pltpu.VMEM((EBLK, 128), _I32),
        pltpu.VMEM((EBLK, 128), _I32),
        pltpu.VMEM((128, C), _F32),
        pltpu.VMEM_SHARED((n_zpad, C), _F32),
        pltpu.SemaphoreType.DMA,
    ]

    @functools.partial(
        pl.kernel, mesh=mesh,
        out_type=jax.ShapeDtypeStruct((nc * n_out, C), _F32),
        compiler_params=pltpu.CompilerParams(use_tc_tiling_on_sc=False),
        scratch_types=scratch)
    def k(*refs):
        tabs = refs[:nc]
        (srcA, srcB, dst2d, zrows, out,
         srcA_v, srcB_v, dst_v, buf, acc, sem) = refs[nc:]
        tid = lax.axis_index("s")
        sc = lax.axis_index("c")
        pltpu.sync_copy(dst2d.at[tid], dst_v)
        pltpu.sync_copy(srcA.at[tid], srcA_v)
        if two_src:
            pltpu.sync_copy(srcB.at[tid], srcB_v)
        for ci in range(nc):
            @pl.when(sc == ci % 2)
            def _(ci=ci):
                pltpu.sync_copy(zrows.at[pl.ds(tid * rz, rz)],
                                acc.at[pl.ds(tid * rz, rz)])
                plsc.subcore_barrier()
                tab = tabs[ci]
                s_v = srcB_v if (two_src and ci >= nc // 2) else srcA_v

                def step(j, carry):
                    pltpu.async_copy(tab.at[s_v.at[j]], buf, sem).wait()
                    pltpu.sync_copy(buf, acc.at[dst_v.at[j]], add=True)
                    return carry

                lax.fori_loop(0, EBLK, step, 0)
                plsc.subcore_barrier()
                pltpu.sync_copy(acc.at[pl.ds(tid * ro, ro)],
                                out.at[pl.ds(ci * n_out + tid * ro, ro)])
                plsc.subcore_barrier()

    return k


def _prop(tabs, srcA, srcB, dst, n_dst, two_src=False):
    nc = len(tabs)
    C = tabs[0].shape[1]
    e_pad = dst.shape[0]
    n_out = -(-n_dst // 128) * 128
    k = _make_prop_kernel(nc, C, e_pad, n_dst, two_src)
    eblk = e_pad // 2048
    srcA2 = srcA.reshape(16, eblk, 128)
    srcB2 = srcB.reshape(16, eblk, 128)
    dst2 = dst.reshape(16, eblk, 128)
    zrows = jnp.zeros((n_out if n_dst < n_out else n_out + 128, C), _F32)
    raw = k(*tabs, srcA2, srcB2, dst2, zrows)
    return [raw[i * n_out:i * n_out + n_dst] for i in range(nc)]


@functools.lru_cache(maxsize=None)
def _make_count_kernel(e_pad, n_dst):
    EBLK = e_pad // 4096
    n_out = -(-n_dst // 128) * 128
    n_zpad = n_out if n_dst < n_out else n_out + 128
    rz = n_zpad // 16
    ro = n_out // 16
    mesh = plsc.VectorSubcoreMesh(core_axis_name="c", subcore_axis_name="s")
    scratch = [
        pltpu.VMEM((EBLK, 128), _I32),
        pltpu.VMEM((128, 16), _F32),
        pltpu.VMEM_SHARED((n_zpad, 16), _F32),
    ]

    @functools.partial(
        pl.kernel, mesh=mesh,
        out_type=jax.ShapeDtypeStruct((2 * n_out, 16), _F32),
        compiler_params=pltpu.CompilerParams(use_tc_tiling_on_sc=False),
        scratch_types=scratch)
    def k(dst2d, zrows, ones_hbm, out, dst_v, buf, acc):
        tid = lax.axis_index("s")
        sc = lax.axis_index("c")
        g = sc * 16 + tid
        pltpu.sync_copy(dst2d.at[g], dst_v)
        pltpu.sync_copy(ones_hbm, buf)
        pltpu.sync_copy(zrows.at[pl.ds(tid * rz, rz)],
                        acc.at[pl.ds(tid * rz, rz)])
        plsc.subcore_barrier()

        def step(j, carry):
            pltpu.sync_copy(buf, acc.at[dst_v.at[j]], add=True)
            return carry

        lax.fori_loop(0, EBLK, step, 0)
        plsc.subcore_barrier()
        pltpu.sync_copy(acc.at[pl.ds(tid * ro, ro)],
                        out.at[pl.ds(sc * n_out + tid * ro, ro)])

    return k


def _seg_count(dst, n_dst):
    E = dst.shape[0]
    e_pad = -(-E // 4096) * 4096
    dstp = jnp.concatenate([dst, jnp.full((e_pad - E,), n_dst, _I32)])
    n_out = -(-n_dst // 128) * 128
    k = _make_count_kernel(e_pad, n_dst)
    ones = jnp.ones((128, 16), _F32)
    zrows = jnp.zeros((n_out if n_dst < n_out else n_out + 128, 16), _F32)
    po = k(dstp.reshape(32, e_pad // 4096, 128), zrows, ones)
    return po[:n_dst, 0] + po[n_out:n_out + n_dst, 0]


def _pad_e(x, e_pad, fill):
    return jnp.concatenate([x, jnp.full((e_pad - x.shape[0],), fill, _I32)])


def _chunks(x, C):
    n, d = x.shape
    return [x[:, i * C:(i + 1) * C] for i in range(d // C)]


def _convkb_body(h_ref, r_ref, t_ref, cw_ref, cb_ref, fc2_ref, o_ref):
    h = h_ref[...]
    r = r_ref[...]
    t = t_ref[...]
    acc = jnp.zeros_like(h)
    for o in range(50):
        co = jax.nn.relu(cw_ref[o, 0] * h + cw_ref[o, 1] * r + cw_ref[o, 2] * t
                         + cb_ref[o])
        acc = acc + co * fc2_ref[o, :][None, :]
    o_ref[...] = jnp.sum(acc, axis=1, keepdims=True)


def _convkb(h, r, t, conv_w, conv_b, fc2):
    blk = 1024
    return pl.pallas_call(
        _convkb_body,
        grid=(B // blk,),
        in_specs=[
            pl.BlockSpec((blk, D), lambda i: (i, 0)),
            pl.BlockSpec((blk, D), lambda i: (i, 0)),
            pl.BlockSpec((blk, D), lambda i: (i, 0)),
            pl.BlockSpec((50, 3), lambda i: (0, 0), memory_space=pltpu.SMEM),
            pl.BlockSpec((50,), lambda i: (0,), memory_space=pltpu.SMEM),
            pl.BlockSpec((50, D), lambda i: (0, 0)),
        ],
        out_specs=pl.BlockSpec((blk, 1), lambda i: (i, 0)),
        out_shape=jax.ShapeDtypeStruct((B, 1), jnp.float32),
    )(h, r, t, conv_w, conv_b, fc2)


def kernel(*args):
    with jax.default_matmul_precision("float32"):
        return _kernel_impl(*args)


def _kernel_impl(entity_embeddings, relation_embeddings, sg1_W1, sg1_W2, sg2_W1,
                 sg2_W2, le_W, le_b, leo_W, leo_b, dgi_W, dgi_b, dgi_Wd,
                 conv_w, conv_b, fc_w, fc_b,
                 b_x, b_node_graph_index, b_edge_index, big_edge_index,
                 batch_inputs, shuf_idx):
    ent = entity_embeddings / (jnp.linalg.norm(entity_embeddings, axis=1,
                                               keepdims=True) + 1e-12)
    rel = relation_embeddings / (jnp.linalg.norm(relation_embeddings, axis=1,
                                                 keepdims=True) + 1e-12)

    b_x = b_x.astype(_I32)
    brel = b_node_graph_index.astype(_I32)
    src = b_edge_index[0].astype(_I32)
    dst = b_edge_index[1].astype(_I32)
    gsrc = big_edge_index[0].astype(_I32)
    gdst = big_edge_index[1].astype(_I32)
    shuf_idx = shuf_idx.astype(_I32)

    deg_b = jnp.maximum(_seg_count(dst, BN), 1.0)
    deg_g = jnp.maximum(_seg_count(gdst, N_ENT), 1.0)
    cnt = _seg_count(b_x, N_ENT)
    S = cnt / (cnt + 1e-16)
    inv_cnt = 1.0 / jnp.maximum(cnt, 1.0)

    eb_pad = -(-EB // 2048) * 2048
    src_p = _pad_e(src, eb_pad, 0)
    dst_p = _pad_e(dst, eb_pad, BN)
    gsrc_p = _pad_e(gsrc, eb_pad, 0)
    gdst_p = _pad_e(gdst, eb_pad, N_ENT)

    P = ent @ sg1_W1[:D]
    Rr = rel @ sg1_W1[D:]
    sx = jnp.take(shuf_idx, b_x)
    BNP = BN + 480
    ar = jnp.arange(BN, dtype=_I32)
    dum = jnp.full((BNP - BN,), BNP, _I32)
    z480 = jnp.zeros((BNP - BN,), _I32)
    dst_asm = jnp.concatenate([ar, dum, ar, dum])
    srcA_asm = jnp.concatenate([b_x, z480, N_ENT + brel, z480])
    srcB_asm = jnp.concatenate([sx, z480, N_ENT + brel, z480])
    Tc = [jnp.concatenate([pc, rc], axis=0)
          for pc, rc in zip(_chunks(P, 64), _chunks(Rr, 64))]
    xwc = _prop(Tc + Tc, srcA_asm, srcB_asm, dst_asm, BNP, two_src=True)

    p1c = _prop(xwc, src_p, src_p, dst_p, BN)
    h1 = jax.nn.relu(jnp.concatenate(p1c[:4], axis=1) / deg_b[:, None])
    h2 = jax.nn.relu(jnp.concatenate(p1c[4:], axis=1) / deg_b[:, None])
    W2p = jnp.pad(sg1_W2, ((0, 0), (0, 40)))
    hw = jnp.concatenate([h1 @ W2p, h2 @ W2p], axis=1)

    g_raw = _prop(_chunks(hw, 80), src_p, src_p, dst_p, BN)
    g12c = [c / deg_b[:, None] for c in g_raw]

    arp = _pad_e(ar, BNP, 0)
    bxp = _pad_e(b_x, BNP, N_ENT)
    o_raw = _prop(g12c, arp, arp, bxp, N_ENT)
    g1 = jnp.concatenate(g12c[:3], axis=1)[:N_ENT, :D]
    g2 = jnp.concatenate(g12c[3:], axis=1)[:N_ENT, :D]
    o1 = jnp.concatenate(o_raw[:3], axis=1)[:, :D] * inv_cnt[:, None]
    o2 = jnp.concatenate(o_raw[3:], axis=1)[:, :D] * inv_cnt[:, None]

    def finish(gh, out, base):
        return S[:, None] * (gh @ leo_W[:D] + out @ leo_W[D:]) + leo_b + base

    ec = finish(g1, o1, ent)
    ec_ = finish(g2, o2, jnp.take(ent, shuf_idx, axis=0))

    z40 = jnp.zeros((N_ENT, 40), _F32)
    e12 = jnp.concatenate([ec, z40, ec_, z40], axis=1)
    p12_raw = _prop(_chunks(e12, 80), gsrc_p, gsrc_p, gdst_p, N_ENT)
    p12c = [c / deg_g[:, None] for c in p12_raw]
    pb1 = jnp.concatenate(p12c[:3], axis=1)[:, :D]
    pb2 = jnp.concatenate(p12c[3:], axis=1)[:, :D]
    W2gp = jnp.pad(sg2_W2, ((0, 0), (0, 40)))
    hbw = jnp.concatenate([jax.nn.relu(pb1 @ sg2_W1) @ W2gp,
                           jax.nn.relu(pb2 @ sg2_W1) @ W2gp], axis=1)
    eg_raw = _prop(_chunks(hbw, 80), gsrc_p, gsrc_p, gdst_p, N_ENT)
    eg = jnp.concatenate(eg_raw[:3], axis=1)[:, :D] / deg_g[:, None]
    eg_ = jnp.concatenate(eg_raw[3:], axis=1)[:, :D] / deg_g[:, None]

    def dgi(h1, h2):
        e1 = jax.nn.relu(h1 @ dgi_W + dgi_b)
        e2 = jax.nn.relu(h2 @ dgi_W + dgi_b)
        c = jax.nn.sigmoid(jnp.mean(e1, axis=0))
        v = dgi_Wd @ c
        return jnp.concatenate([e1 @ v, e2 @ v])[None, :]

    local_logits = dgi(ec, ec_)
    global_logits = dgi(eg, eg_)

    h = jnp.take(ec, batch_inputs[:, 0], axis=0)
    r = jnp.take(rel, batch_inputs[:, 1], axis=0)
    t = jnp.take(ec, batch_inputs[:, 2], axis=0)
    out_conv = _convkb(h, r, t, conv_w, conv_b, fc_w.reshape(50, D)) + fc_b
    return (out_conv, local_logits, global_logits)
